# Initial kernel scaffold; baseline (speedup 1.0000x reference)
import jax
import jax.numpy as jnp
from jax.experimental import pallas as pl


def kernel(x, edge_index, edge_attr, eW1, eb1, eW2, eb2, eW3, eb3, e_gamma, e_beta, nW1, nb1, nW2, nb2, nW3, nb3, n_gamma, n_beta):
    raise NotImplementedError("write your pallas kernel here")


# speedup vs baseline: 7.6376x; 7.6376x over previous
import functools

import jax
import jax.numpy as jnp
from jax import lax
from jax.experimental import pallas as pl
from jax.experimental.pallas import tpu as pltpu
from jax.experimental.pallas import tpu_sc as plsc

N_NODES = 10000
N_EDGES = 320000
ND = 128
ED = 16

NC, NS = 2, 16
NW = NC * NS
EPW = N_EDGES // NW
CH = 100
NCH = EPW // CH
NB = 4
NG = NCH // NB
RPT = N_NODES // NS

_EPS = 1e-5


def _pq_body(x_ref, w_ref, p_ref, q_ref):
    pq = jnp.dot(x_ref[...], w_ref[...], preferred_element_type=jnp.float32)
    p_ref[...] = pq[:, :ED]
    q_ref[...] = pq[:, ED:]


def _project_pq(x, w_pq):
    return pl.pallas_call(
        _pq_body,
        grid=(10,),
        in_specs=[
            pl.BlockSpec((1000, ND), lambda m: (m, 0)),
            pl.BlockSpec((ND, 2 * ED), lambda m: (0, 0)),
        ],
        out_specs=[
            pl.BlockSpec((1000, ED), lambda m: (m, 0)),
            pl.BlockSpec((1000, ED), lambda m: (m, 0)),
        ],
        out_shape=[
            jax.ShapeDtypeStruct((N_NODES, ED), jnp.float32),
            jax.ShapeDtypeStruct((N_NODES, ED), jnp.float32),
        ],
    )(x, w_pq)


def _gather_body(p_hbm, q_hbm, i2_hbm, j2_hbm, out_hbm,
                 idxi_v, idxj_v, ra, rb, ro,
                 sa0, sa1, sa2, sa3, sb0, sb1, sb2, sb3,
                 so0, so1, so2, so3):
    sas = (sa0, sa1, sa2, sa3)
    sbs = (sb0, sb1, sb2, sb3)
    sos = (so0, so1, so2, so3)
    wid = lax.axis_index("s") * NC + lax.axis_index("c")
    pltpu.sync_copy(i2_hbm.at[pl.ds(wid * NCH, NCH)], idxi_v)
    pltpu.sync_copy(j2_hbm.at[pl.ds(wid * NCH, NCH)], idxj_v)
    ebase = wid * EPW

    def issue(t, b):
        pltpu.async_copy(p_hbm.at[idxi_v.at[t]], ra.at[b], sas[b])
        pltpu.async_copy(q_hbm.at[idxj_v.at[t]], rb.at[b], sbs[b])

    def drain(d, b, g):
        pltpu.make_async_copy(p_hbm.at[pl.ds(0, CH)], ra.at[b], sas[b]).wait()
        pltpu.make_async_copy(q_hbm.at[pl.ds(0, CH)], rb.at[b], sbs[b]).wait()

        @pl.when(g >= 1)
        def _():
            pltpu.make_async_copy(
                ro.at[b], out_hbm.at[pl.ds(0, CH)], sos[b]).wait()
        rav, rbv, rov = ra.at[b], rb.at[b], ro.at[b]

        def add_row(r, c):
            rov[r] = rav[r] + rbv[r]
            return c
        lax.fori_loop(0, CH, add_row, 0, unroll=4)
        pltpu.async_copy(ro.at[b], out_hbm.at[pl.ds(ebase + d * CH, CH)],
                         sos[b])

    for b in range(NB - 1):
        issue(b, b)

    def body(g, c):
        for u in range(NB):
            t = g * NB + u
            tn = t + (NB - 1)
            bn = (u + NB - 1) % NB

            @pl.when(tn < NCH)
            def _():
                issue(tn, bn)
            drain(t, u, g)
        return c

    lax.fori_loop(0, NG, body, 0)
    for b in range(NB):
        pltpu.make_async_copy(ro.at[b], out_hbm.at[pl.ds(0, CH)],
                              sos[b]).wait()


def _gather_pq(p, q, i2, j2):
    f = pl.kernel(
        _gather_body,
        out_type=jax.ShapeDtypeStruct((N_EDGES, ED), jnp.float32),
        mesh=plsc.VectorSubcoreMesh(
            core_axis_name="c", subcore_axis_name="s",
            num_cores=NC, num_subcores=NS),
        scratch_types=[
            pltpu.VMEM((NCH, CH), jnp.int32),
            pltpu.VMEM((NCH, CH), jnp.int32),
            pltpu.VMEM((NB, CH, ED), jnp.float32),
            pltpu.VMEM((NB, CH, ED), jnp.float32),
            pltpu.VMEM((NB, CH, ED), jnp.float32),
        ] + [pltpu.SemaphoreType.DMA] * (3 * NB),
        compiler_params=pltpu.CompilerParams(use_tc_tiling_on_sc=False),
    )
    return f(p, q, i2, j2)


def _edge_body(g_ref, ea_ref, we_ref, m_ref, b1_ref, w2_ref, b2_ref,
               w3_ref, b3_ref, gam_ref, bet_ref, out_ref):
    ea = ea_ref[...]
    t = g_ref[...] + jnp.dot(ea, we_ref[...],
                             preferred_element_type=jnp.float32) + b1_ref[...]
    t = t * jax.nn.sigmoid(t)
    t = jnp.dot(t, w2_ref[...], preferred_element_type=jnp.float32) + b2_ref[...]
    t = t * jax.nn.sigmoid(t)
    t = jnp.dot(t, w3_ref[...], preferred_element_type=jnp.float32) + b3_ref[...]
    mu = jnp.dot(t, m_ref[...], preferred_element_type=jnp.float32)
    d = t - mu
    var = jnp.dot(d * d, m_ref[...], preferred_element_type=jnp.float32)
    out_ref[...] = ea + d * lax.rsqrt(var + _EPS) * gam_ref[...] + bet_ref[...]


def _edge_mlp(g2, ea2, we_b, m_b, b1_t, w2_b, b2_t, w3_b, b3_t, gam_t, bet_t):
    rows = N_EDGES * ED // ND
    blk = 2000
    full = pl.BlockSpec((ND, ND), lambda m: (0, 0))
    vec = pl.BlockSpec((1, ND), lambda m: (0, 0))
    return pl.pallas_call(
        _edge_body,
        grid=(rows // blk,),
        in_specs=[
            pl.BlockSpec((blk, ND), lambda m: (m, 0)),
            pl.BlockSpec((blk, ND), lambda m: (m, 0)),
            full, full, vec, full, vec, full, vec, vec, vec,
        ],
        out_specs=pl.BlockSpec((blk, ND), lambda m: (m, 0)),
        out_shape=jax.ShapeDtypeStruct((rows, ND), jnp.float32),
    )(g2, ea2, we_b, m_b, b1_t, w2_b, b2_t, w3_b, b3_t, gam_t, bet_t)


def _scatter_body(e_hbm, j2_hbm, out_hbm,
                  idxj_v, re, zbuf, acc,
                  se0, se1, se2, se3, ss0, ss1, ss2, ss3):
    ses = (se0, se1, se2, se3)
    sss = (ss0, ss1, ss2, ss3)
    cid = lax.axis_index("c")
    sid = lax.axis_index("s")
    wid = sid * NC + cid
    ebase = wid * EPW

    def zb(r, c):
        zbuf[r] = jnp.zeros((ED,), jnp.float32)
        return c
    lax.fori_loop(0, 125, zb, 0, unroll=4)
    for k in range(RPT // 125):
        pltpu.sync_copy(zbuf, acc.at[pl.ds(sid * RPT + k * 125, 125)])
    plsc.subcore_barrier()

    pltpu.sync_copy(j2_hbm.at[pl.ds(wid * NCH, NCH)], idxj_v)

    def issue(t, b):
        pltpu.async_copy(e_hbm.at[pl.ds(ebase + t * CH, CH)], re.at[b],
                         ses[b])

    def drain(d, b):
        pltpu.make_async_copy(e_hbm.at[pl.ds(0, CH)], re.at[b], ses[b]).wait()
        pltpu.async_copy(re.at[b], acc.at[idxj_v.at[d]], sss[b], add=True)

    for b in range(NB - 1):
        issue(b, b)

    def body(g, c):
        for u in range(NB):
            t = g * NB + u
            tn = t + (NB - 1)
            bn = (u + NB - 1) % NB

            @pl.when(tn < NCH)
            def _():
                @pl.when(tn >= NB)
                def _():
                    pltpu.make_async_copy(
                        re.at[bn], acc.at[idxj_v.at[0]], sss[bn]).wait()
                issue(tn, bn)
            drain(t, u)
        return c
    lax.fori_loop(0, NG, body, 0)
    for b in range(NB):
        pltpu.make_async_copy(re.at[b], acc.at[idxj_v.at[0]], sss[b]).wait()

    plsc.subcore_barrier()
    pltpu.sync_copy(acc.at[pl.ds(sid * RPT, RPT)],
                    out_hbm.at[cid, pl.ds(sid * RPT, RPT)])


def _scatter_add(e_new, j2):
    f = pl.kernel(
        _scatter_body,
        out_type=jax.ShapeDtypeStruct((NC, N_NODES, ED), jnp.float32),
        mesh=plsc.VectorSubcoreMesh(
            core_axis_name="c", subcore_axis_name="s",
            num_cores=NC, num_subcores=NS),
        scratch_types=[
            pltpu.VMEM((NCH, CH), jnp.int32),
            pltpu.VMEM((NB, CH, ED), jnp.float32),
            pltpu.VMEM((125, ED), jnp.float32),
            pltpu.VMEM_SHARED((N_NODES, ED), jnp.float32),
        ] + [pltpu.SemaphoreType.DMA] * (2 * NB),
        compiler_params=pltpu.CompilerParams(use_tc_tiling_on_sc=False),
    )
    return f(e_new, j2)


def _node_body(x_ref, a0_ref, a1_ref, w1a_ref, w1b_ref, b1_ref,
               w2_ref, b2_ref, w3_ref, b3_ref, gam_ref, bet_ref, out_ref):
    x = x_ref[...]
    agg = a0_ref[...] + a1_ref[...]
    t = (jnp.dot(x, w1a_ref[...], preferred_element_type=jnp.float32)
         + jnp.dot(agg, w1b_ref[...], preferred_element_type=jnp.float32)
         + b1_ref[...])
    t = t * jax.nn.sigmoid(t)
    t = jnp.dot(t, w2_ref[...], preferred_element_type=jnp.float32) + b2_ref[...]
    t = t * jax.nn.sigmoid(t)
    t = jnp.dot(t, w3_ref[...], preferred_element_type=jnp.float32) + b3_ref[...]
    mu = jnp.mean(t, axis=-1, keepdims=True)
    d = t - mu
    var = jnp.mean(d * d, axis=-1, keepdims=True)
    out_ref[...] = x + d * lax.rsqrt(var + _EPS) * gam_ref[...] + bet_ref[...]


def _node_mlp(x, a0, a1, w1a, w1b, b1, w2, b2, w3, b3, gam, bet):
    blk = 1000
    full = pl.BlockSpec((ND, ND), lambda m: (0, 0))
    vec = pl.BlockSpec((1, ND), lambda m: (0, 0))
    return pl.pallas_call(
        _node_body,
        grid=(N_NODES // blk,),
        in_specs=[
            pl.BlockSpec((blk, ND), lambda m: (m, 0)),
            pl.BlockSpec((blk, ED), lambda m: (m, 0)),
            pl.BlockSpec((blk, ED), lambda m: (m, 0)),
            full,
            pl.BlockSpec((ED, ND), lambda m: (0, 0)),
            vec, full, vec, full, vec, vec, vec,
        ],
        out_specs=pl.BlockSpec((blk, ND), lambda m: (m, 0)),
        out_shape=jax.ShapeDtypeStruct((N_NODES, ND), jnp.float32),
    )(x, a0, a1, w1a, w1b, b1, w2, b2, w3, b3, gam, bet)


def _bd8(w):
    return jnp.kron(jnp.eye(8, dtype=w.dtype), w)


def kernel(x, edge_index, edge_attr,
           eW1, eb1, eW2, eb2, eW3, eb3, e_gamma, e_beta,
           nW1, nb1, nW2, nb2, nW3, nb3, n_gamma, n_beta):
    i2 = edge_index[0].astype(jnp.int32).reshape(NW * NCH, CH)
    j2 = edge_index[1].astype(jnp.int32).reshape(NW * NCH, CH)

    w_pq = jnp.concatenate([eW1[:ND], eW1[ND:2 * ND]], axis=1)
    p, q = _project_pq(x, w_pq)

    g = _gather_pq(p, q, i2, j2)

    rows = N_EDGES * ED // ND
    tile8 = lambda v: jnp.tile(v, 8).reshape(1, ND)
    e_new2 = _edge_mlp(
        g.reshape(rows, ND), edge_attr.reshape(rows, ND),
        _bd8(eW1[2 * ND:]), _bd8(jnp.full((ED, ED), 1.0 / ED, jnp.float32)),
        tile8(eb1), _bd8(eW2), tile8(eb2), _bd8(eW3), tile8(eb3),
        tile8(e_gamma), tile8(e_beta))
    e_new = e_new2.reshape(N_EDGES, ED)

    aggp = _scatter_add(e_new, j2)

    x_new = _node_mlp(
        x, aggp[0], aggp[1],
        nW1[:ND], nW1[ND:], nb1.reshape(1, ND),
        nW2, nb2.reshape(1, ND), nW3, nb3.reshape(1, ND),
        n_gamma.reshape(1, ND), n_beta.reshape(1, ND))

    return (x_new, e_new)
